# baseline (speedup 1.0000x reference)
import functools

import jax
import jax.numpy as jnp
from jax import lax
from jax.experimental import pallas as pl
from jax.experimental.pallas import tpu as pltpu
from jax.experimental.pallas import tpu_sc as plsc

NUM_CORES = 2
NUM_SUBCORES = 16
NUM_WORKERS = NUM_CORES * NUM_SUBCORES
CHUNK = 128
DIM = 64


def _make_kernel(n_chunks: int):
    mesh = plsc.VectorSubcoreMesh(core_axis_name="c", subcore_axis_name="s")

    @functools.partial(
        pl.kernel,
        out_type=jax.ShapeDtypeStruct((NUM_WORKERS, n_chunks, CHUNK, DIM),
                                      jnp.float32),
        mesh=mesh,
        scratch_types=[
            pltpu.VMEM((n_chunks, CHUNK), jnp.int32),
            pltpu.VMEM((2, CHUNK, DIM), jnp.float32),
            pltpu.SemaphoreType.DMA,
        ],
        compiler_params=pltpu.CompilerParams(use_tc_tiling_on_sc=False),
    )
    def k(x_hbm, w_hbm, out_hbm, idx_v, rows_v, gsem):
        wid = lax.axis_index("s") * NUM_CORES + lax.axis_index("c")
        pltpu.sync_copy(x_hbm.at[wid], idx_v)

        @pl.loop(0, n_chunks)
        def _(j):
            pltpu.async_copy(w_hbm.at[idx_v.at[j]], rows_v.at[0], gsem).wait()
            pltpu.sync_copy(rows_v.at[0], out_hbm.at[wid, j])

    return k


def kernel(x, weight):
    b = x.size
    n_chunks = b // (NUM_WORKERS * CHUNK)
    x3 = x.reshape(NUM_WORKERS, n_chunks, CHUNK).astype(jnp.int32)
    out = _make_kernel(n_chunks)(x3, weight)
    return out.reshape(x.shape + (DIM,))


# speedup vs baseline: 1.1096x; 1.1096x over previous
import functools

import jax
import jax.numpy as jnp
from jax import lax
from jax.experimental import pallas as pl
from jax.experimental.pallas import tpu as pltpu
from jax.experimental.pallas import tpu_sc as plsc

NUM_CORES = 2
NUM_SUBCORES = 16
NUM_WORKERS = NUM_CORES * NUM_SUBCORES
CHUNK = 128
DIM = 64
NBUF = 8


def _make_kernel(n_chunks: int):
    mesh = plsc.VectorSubcoreMesh(core_axis_name="c", subcore_axis_name="s")

    @functools.partial(
        pl.kernel,
        out_type=jax.ShapeDtypeStruct((NUM_WORKERS, n_chunks, CHUNK, DIM),
                                      jnp.float32),
        mesh=mesh,
        scratch_types=[
            pltpu.VMEM((n_chunks, CHUNK), jnp.int32),
            pltpu.VMEM((NBUF, CHUNK, DIM), jnp.float32),
            pltpu.SemaphoreType.DMA,
            pltpu.SemaphoreType.DMA,
        ],
        compiler_params=pltpu.CompilerParams(use_tc_tiling_on_sc=False),
    )
    def k(x_hbm, w_hbm, out_hbm, idx_v, bufs, gsem, wsem):
        wid = lax.axis_index("s") * NUM_CORES + lax.axis_index("c")
        pltpu.sync_copy(x_hbm.at[wid], idx_v)

        for t in range(NBUF - 1):
            pltpu.async_copy(w_hbm.at[idx_v.at[t]], bufs.at[t], gsem)

        @pl.loop(0, n_chunks)
        def _(j):
            s = j % NBUF
            pltpu.make_async_copy(w_hbm.at[pl.ds(0, CHUNK)], bufs.at[s],
                                  gsem).wait()
            pltpu.async_copy(bufs.at[s], out_hbm.at[wid, j], wsem)
            nj = j + NBUF - 1

            @pl.when(nj < n_chunks)
            def _():
                @pl.when(j >= 1)
                def _():
                    pltpu.make_async_copy(bufs.at[0], out_hbm.at[wid, 0],
                                          wsem).wait()

                pltpu.async_copy(w_hbm.at[idx_v.at[nj]], bufs.at[nj % NBUF],
                                 gsem)

        for _ in range(NBUF):
            pltpu.make_async_copy(bufs.at[0], out_hbm.at[wid, 0], wsem).wait()

    return k


def kernel(x, weight):
    b = x.size
    n_chunks = b // (NUM_WORKERS * CHUNK)
    x3 = x.reshape(NUM_WORKERS, n_chunks, CHUNK).astype(jnp.int32)
    out = _make_kernel(n_chunks)(x3, weight)
    return out.reshape(x.shape + (DIM,))


# speedup vs baseline: 1.1667x; 1.0515x over previous
import functools

import jax
import jax.numpy as jnp
from jax import lax
from jax.experimental import pallas as pl
from jax.experimental.pallas import tpu as pltpu
from jax.experimental.pallas import tpu_sc as plsc

NUM_CORES = 2
NUM_SUBCORES = 16
NUM_WORKERS = NUM_CORES * NUM_SUBCORES
CHUNK = 128
DIM = 64
NBUF = 8


def _make_kernel(n_chunks: int):
    mesh = plsc.VectorSubcoreMesh(core_axis_name="c", subcore_axis_name="s")
    n_blocks = NUM_WORKERS * n_chunks

    @functools.partial(
        pl.kernel,
        out_type=jax.ShapeDtypeStruct((n_blocks, CHUNK, DIM), jnp.float32),
        mesh=mesh,
        scratch_types=[
            pltpu.VMEM((n_chunks, CHUNK), jnp.int32),
            pltpu.VMEM((NBUF, CHUNK, DIM), jnp.float32),
            pltpu.SemaphoreType.DMA,
            pltpu.SemaphoreType.DMA,
        ],
        compiler_params=pltpu.CompilerParams(use_tc_tiling_on_sc=False),
    )
    def k(x_hbm, w_hbm, out_hbm, idx_v, bufs, gsem, wsem):
        wid = lax.axis_index("s") * NUM_CORES + lax.axis_index("c")
        base = wid * n_chunks
        pltpu.sync_copy(x_hbm.at[pl.ds(base, n_chunks)], idx_v)

        for t in range(NBUF - 1):
            pltpu.async_copy(w_hbm.at[idx_v.at[t]], bufs.at[t], gsem)

        @pl.loop(0, n_chunks)
        def _(j):
            s = j % NBUF
            pltpu.make_async_copy(w_hbm.at[pl.ds(0, CHUNK)], bufs.at[s],
                                  gsem).wait()
            pltpu.async_copy(bufs.at[s], out_hbm.at[base + j], wsem)
            nj = j + NBUF - 1

            @pl.when(nj < n_chunks)
            def _():
                @pl.when(j >= 1)
                def _():
                    pltpu.make_async_copy(bufs.at[0], out_hbm.at[0],
                                          wsem).wait()

                pltpu.async_copy(w_hbm.at[idx_v.at[nj]], bufs.at[nj % NBUF],
                                 gsem)

        for _ in range(NBUF):
            pltpu.make_async_copy(bufs.at[0], out_hbm.at[0], wsem).wait()

    return k


def kernel(x, weight):
    rows, cols = x.shape
    b = rows * cols
    n_chunks = b // (NUM_WORKERS * CHUNK)
    x2 = x.T.reshape(b // CHUNK, CHUNK).astype(jnp.int32)
    out = _make_kernel(n_chunks)(x2, weight)
    return out.reshape(cols, rows, DIM).transpose(1, 0, 2)
